# Initial kernel scaffold; baseline (speedup 1.0000x reference)
import jax
import jax.numpy as jnp
from jax.experimental import pallas as pl


def kernel(x, token_table, pos_table):
    raise NotImplementedError("write your pallas kernel here")


# speedup vs baseline: 2.1059x; 2.1059x over previous
import functools

import jax
import jax.numpy as jnp
from jax import lax
from jax.experimental import pallas as pl
from jax.experimental.pallas import tpu as pltpu
from jax.experimental.pallas import tpu_sc as plsc

_LANES = 16


def _make_sc_kernel(n_rows, chunk, maxlen, embed):
    nc, ns = 2, 16
    nw = nc * ns
    assert n_rows % nw == 0
    rows_per_w = n_rows // nw
    assert maxlen % chunk == 0 and embed % _LANES == 0
    pos_chunks = maxlen // chunk
    k_sl = embed // _LANES

    mesh = plsc.VectorSubcoreMesh(core_axis_name="c", subcore_axis_name="s")

    @functools.partial(
        pl.kernel,
        mesh=mesh,
        compiler_params=pltpu.CompilerParams(use_tc_tiling_on_sc=False),
        out_type=jax.ShapeDtypeStruct((n_rows * chunk, embed), jnp.float32),
        scratch_types=[
            pltpu.VMEM((rows_per_w, chunk), jnp.int32),
            pltpu.VMEM((maxlen, embed), jnp.float32),
            pltpu.VMEM((chunk, embed), jnp.float32),
            pltpu.SemaphoreType.DMA,
        ],
    )
    def sc_kernel(x_hbm, tab_hbm, pos_hbm, out_hbm, idx_v, pos_v, buf, gsem):
        cid = lax.axis_index("c")
        sid = lax.axis_index("s")
        wid = sid * nc + cid
        base = wid * rows_per_w
        pltpu.sync_copy(x_hbm.at[pl.ds(base, rows_per_w)], idx_v)
        pltpu.sync_copy(pos_hbm, pos_v)

        def chunk_body(j, _):
            pltpu.async_copy(tab_hbm.at[idx_v.at[j]], buf, gsem).wait()
            poff = (j % pos_chunks) * chunk

            def add_body(i, _):
                for k in range(k_sl):
                    s = pl.ds(k * _LANES, _LANES)
                    buf[i, s] = buf[i, s] + pos_v[poff + i, s]
                return 0

            lax.fori_loop(0, chunk, add_body, 0)
            pltpu.sync_copy(buf, out_hbm.at[pl.ds((base + j) * chunk, chunk)])
            return 0

        lax.fori_loop(0, rows_per_w, chunk_body, 0)

    return sc_kernel


def kernel(x, token_table, pos_table):
    batch, maxlen = x.shape
    _, embed = token_table.shape
    chunk = 100
    n_rows = batch * maxlen // chunk
    x2 = x.reshape(n_rows, chunk)
    sc = _make_sc_kernel(n_rows, chunk, maxlen, embed)
    out = sc(x2, token_table, pos_table)
    return out.reshape(batch, maxlen, embed)


# speedup vs baseline: 2.7333x; 1.2979x over previous
import functools

import jax
import jax.numpy as jnp
from jax import lax
from jax.experimental import pallas as pl
from jax.experimental.pallas import tpu as pltpu
from jax.experimental.pallas import tpu_sc as plsc

_LANES = 16


def _make_sc_kernel(n_rows, chunk, maxlen, embed):
    nc, ns = 2, 16
    nw = nc * ns
    assert n_rows % nw == 0
    rows_per_w = n_rows // nw
    assert rows_per_w % 2 == 0
    assert maxlen == 2 * chunk and embed % _LANES == 0
    k_sl = embed // _LANES

    mesh = plsc.VectorSubcoreMesh(core_axis_name="c", subcore_axis_name="s")

    @functools.partial(
        pl.kernel,
        mesh=mesh,
        compiler_params=pltpu.CompilerParams(use_tc_tiling_on_sc=False),
        out_type=jax.ShapeDtypeStruct((n_rows * chunk, embed), jnp.float32),
        scratch_types=[
            pltpu.VMEM((rows_per_w, chunk), jnp.int32),
            pltpu.VMEM((maxlen, embed), jnp.float32),
            [pltpu.VMEM((chunk, embed), jnp.float32) for _ in range(2)],
            [pltpu.VMEM((chunk, embed), jnp.float32) for _ in range(2)],
            [pltpu.SemaphoreType.DMA for _ in range(2)],
            [pltpu.SemaphoreType.DMA for _ in range(2)],
        ],
    )
    def sc_kernel(x_hbm, tab_hbm, pos_hbm, out_hbm, idx_v, pos_v, gbuf, sbuf,
                  gsem, ssem):
        cid = lax.axis_index("c")
        sid = lax.axis_index("s")
        wid = sid * nc + cid
        base = wid * rows_per_w
        pltpu.sync_copy(x_hbm.at[pl.ds(base, rows_per_w)], idx_v)
        pltpu.sync_copy(pos_hbm, pos_v)

        def fire_gather(j, b):
            pltpu.async_copy(tab_hbm.at[idx_v.at[j]], gbuf[b], gsem[b])

        def wait_gather(b):
            pltpu.make_async_copy(tab_hbm.at[pl.ds(0, chunk)], gbuf[b],
                                  gsem[b]).wait()

        def fire_store(j, b):
            pltpu.async_copy(sbuf[b], out_hbm.at[pl.ds((base + j) * chunk,
                                                       chunk)], ssem[b])

        def wait_store(b):
            pltpu.make_async_copy(sbuf[b], out_hbm.at[pl.ds(0, chunk)],
                                  ssem[b]).wait()

        def add_pos(b):
            poff = b * chunk

            def add_body(i, _):
                for k in range(k_sl):
                    s = pl.ds(k * _LANES, _LANES)
                    sbuf[b][i, s] = gbuf[b][i, s] + pos_v[poff + i, s]
                return 0

            lax.fori_loop(0, chunk, add_body, 0, unroll=2)

        for b in range(2):
            fire_gather(b, b)
        for b in range(2):
            wait_gather(b)
            add_pos(b)
            fire_gather(b + 2, b)
            fire_store(b, b)

        def chunk_body(t, _):
            j0 = 2 * t
            for b in range(2):
                j = j0 + b
                wait_gather(b)
                wait_store(b)
                add_pos(b)

                @pl.when(j + 2 < rows_per_w)
                def _():
                    fire_gather(j + 2, b)

                fire_store(j, b)
            return 0

        lax.fori_loop(1, rows_per_w // 2, chunk_body, 0, unroll=1)

        for b in range(2):
            wait_store(b)

    return sc_kernel


def kernel(x, token_table, pos_table):
    batch, maxlen = x.shape
    _, embed = token_table.shape
    chunk = maxlen // 2
    n_rows = batch * 2
    x2 = x.reshape(n_rows, chunk)
    sc = _make_sc_kernel(n_rows, chunk, maxlen, embed)
    out = sc(x2, token_table, pos_table)
    return out.reshape(batch, maxlen, embed)


# speedup vs baseline: 2.7345x; 1.0004x over previous
import functools

import jax
import jax.numpy as jnp
from jax import lax
from jax.experimental import pallas as pl
from jax.experimental.pallas import tpu as pltpu
from jax.experimental.pallas import tpu_sc as plsc

_LANES = 16


def _make_sc_kernel(n_rows, chunk, maxlen, embed):
    nc, ns = 2, 16
    nw = nc * ns
    assert n_rows % nw == 0
    rows_per_w = n_rows // nw
    assert rows_per_w % 2 == 0
    assert maxlen == 2 * chunk and embed % _LANES == 0
    k_sl = embed // _LANES

    mesh = plsc.VectorSubcoreMesh(core_axis_name="c", subcore_axis_name="s")

    @functools.partial(
        pl.kernel,
        mesh=mesh,
        compiler_params=pltpu.CompilerParams(use_tc_tiling_on_sc=False),
        out_type=jax.ShapeDtypeStruct((n_rows // 2, maxlen, embed),
                                      jnp.float32),
        scratch_types=[
            pltpu.VMEM((rows_per_w, chunk), jnp.int32),
            pltpu.VMEM((maxlen, embed), jnp.float32),
            [pltpu.VMEM((chunk, embed), jnp.float32) for _ in range(2)],
            [pltpu.VMEM((chunk, embed), jnp.float32) for _ in range(2)],
            [pltpu.SemaphoreType.DMA for _ in range(2)],
            [pltpu.SemaphoreType.DMA for _ in range(2)],
        ],
    )
    def sc_kernel(x_hbm, tab_hbm, pos_hbm, out_hbm, idx_v, pos_v, gbuf, sbuf,
                  gsem, ssem):
        cid = lax.axis_index("c")
        sid = lax.axis_index("s")
        wid = sid * nc + cid
        base = wid * rows_per_w
        pltpu.sync_copy(x_hbm.at[pl.ds(base, rows_per_w)], idx_v)
        pltpu.sync_copy(pos_hbm, pos_v)

        def fire_gather(j, b):
            pltpu.async_copy(tab_hbm.at[idx_v.at[j]], gbuf[b], gsem[b])

        def wait_gather(b):
            pltpu.make_async_copy(tab_hbm.at[pl.ds(0, chunk)], gbuf[b],
                                  gsem[b]).wait()

        def fire_store(j, b):
            row = (base + j) // 2
            pltpu.async_copy(sbuf[b],
                             out_hbm.at[row, pl.ds(b * chunk, chunk)],
                             ssem[b])

        def wait_store(b):
            pltpu.make_async_copy(sbuf[b], out_hbm.at[0, pl.ds(0, chunk)],
                                  ssem[b]).wait()

        def add_pos(b):
            poff = b * chunk

            def add_body(i, _):
                for k in range(k_sl):
                    s = pl.ds(k * _LANES, _LANES)
                    sbuf[b][i, s] = gbuf[b][i, s] + pos_v[poff + i, s]
                return 0

            lax.fori_loop(0, chunk, add_body, 0, unroll=2)

        for b in range(2):
            fire_gather(b, b)
        for b in range(2):
            wait_gather(b)
            add_pos(b)
            fire_gather(b + 2, b)
            fire_store(b, b)

        def chunk_body(t, _):
            j0 = 2 * t
            for b in range(2):
                j = j0 + b
                wait_gather(b)
                wait_store(b)
                add_pos(b)

                @pl.when(j + 2 < rows_per_w)
                def _():
                    fire_gather(j + 2, b)

                fire_store(j, b)
            return 0

        lax.fori_loop(1, rows_per_w // 2, chunk_body, 0, unroll=1)

        for b in range(2):
            wait_store(b)

    return sc_kernel


def kernel(x, token_table, pos_table):
    batch, maxlen = x.shape
    _, embed = token_table.shape
    chunk = maxlen // 2
    n_rows = batch * 2
    x2 = x.reshape(n_rows, chunk)
    sc = _make_sc_kernel(n_rows, chunk, maxlen, embed)
    return sc(x2, token_table, pos_table)
